# Initial kernel scaffold; baseline (speedup 1.0000x reference)
import jax
import jax.numpy as jnp
from jax.experimental import pallas as pl


def kernel(x, pos, edge_index, edge_attr, We1, be1, We2, be2, Wn1, bn1, Wn2, bn2):
    raise NotImplementedError("write your pallas kernel here")


# speedup vs baseline: 2.3227x; 2.3227x over previous
import functools

import jax
import jax.numpy as jnp
from jax import lax
from jax.experimental import pallas as pl
from jax.experimental.pallas import tpu as pltpu
from jax.experimental.pallas import tpu_sc as plsc

N = 10000
E = 320000
H = 128
ED = 16
L = 2

NC = 2
NS = 16
NW = NC * NS
EPW = E // NW
CH = 80
NCH = EPW // CH
NPAD = 10240
NPS = NPAD // NS

F32 = jnp.float32
I32 = jnp.int32


def _sc_mesh():
    return plsc.VectorSubcoreMesh(
        core_axis_name="c", subcore_axis_name="s",
        num_cores=NC, num_subcores=NS)


def _gather_pass(a, b, src, dst, px, py, pz, wd2):
    @functools.partial(
        pl.kernel,
        out_type=jax.ShapeDtypeStruct((E, H), F32),
        mesh=_sc_mesh(),
        scratch_types=[
            pltpu.VMEM((EPW,), I32),
            pltpu.VMEM((EPW,), I32),
            pltpu.VMEM((N,), F32),
            pltpu.VMEM((N,), F32),
            pltpu.VMEM((N,), F32),
            pltpu.VMEM((H,), F32),
            pltpu.VMEM((CH,), F32),
            pltpu.VMEM((CH, H), F32),
            pltpu.VMEM((CH, H), F32),
            pltpu.VMEM((CH, H), F32),
            pltpu.SemaphoreType.DMA,
            pltpu.SemaphoreType.DMA,
        ],
        compiler_params=pltpu.CompilerParams(needs_layout_passes=False),
    )
    def k(a_h, b_h, src_h, dst_h, px_h, py_h, pz_h, w_h, out_h,
          src_v, dst_v, px_v, py_v, pz_v, w_v, d2_v, a_v, b_v, g_v, s1, s2):
        cid = lax.axis_index("c")
        sid = lax.axis_index("s")
        base = (cid * NS + sid) * EPW
        pltpu.sync_copy(src_h.at[pl.ds(base, EPW)], src_v)
        pltpu.sync_copy(dst_h.at[pl.ds(base, EPW)], dst_v)
        pltpu.sync_copy(px_h, px_v)
        pltpu.sync_copy(py_h, py_v)
        pltpu.sync_copy(pz_h, pz_v)
        pltpu.sync_copy(w_h, w_v)

        def chunk(kk, carry):
            o = kk * CH
            c1 = pltpu.async_copy(a_h.at[src_v.at[pl.ds(o, CH)]], a_v, s1)
            c2 = pltpu.async_copy(b_h.at[dst_v.at[pl.ds(o, CH)]], b_v, s2)
            for q in range(CH // 16):
                si = src_v[pl.ds(o + q * 16, 16)]
                di = dst_v[pl.ds(o + q * 16, 16)]
                rx = plsc.load_gather(px_v, [si]) - plsc.load_gather(px_v, [di])
                ry = plsc.load_gather(py_v, [si]) - plsc.load_gather(py_v, [di])
                rz = plsc.load_gather(pz_v, [si]) - plsc.load_gather(pz_v, [di])
                d2_v[pl.ds(q * 16, 16)] = rx * rx + ry * ry + rz * rz
            c1.wait()
            c2.wait()
            wrow = [w_v[pl.ds(j * 16, 16)] for j in range(H // 16)]

            def grp(q, c):
                r0 = q * 16
                d2g = d2_v[pl.ds(r0, 16)]
                for i in range(16):
                    r = r0 + i
                    d2s = d2g[i]
                    for j in range(H // 16):
                        sl = pl.ds(j * 16, 16)
                        g_v[r, sl] = a_v[r, sl] + b_v[r, sl] + d2s * wrow[j]
                return c

            lax.fori_loop(0, CH // 16, grp, 0)
            pltpu.sync_copy(g_v, out_h.at[pl.ds(base + o, CH)])
            return carry

        lax.fori_loop(0, NCH, chunk, 0)

    return k(a, b, src, dst, px, py, pz, wd2)


def _scatter_pass(m2, dst, zeros_n):
    @functools.partial(
        pl.kernel,
        out_type=jax.ShapeDtypeStruct((NC, NPAD, H), F32),
        mesh=_sc_mesh(),
        scratch_types=[
            pltpu.VMEM((CH, H), F32),
            pltpu.VMEM((CH,), I32),
            pltpu.VMEM_SHARED((NPAD, H), F32),
        ],
    )
    def k(m2_h, dst_h, zz_h, out_h, m_v, idx_v, agg_sh):
        cid = lax.axis_index("c")
        sid = lax.axis_index("s")
        base = (cid * NS + sid) * EPW
        pltpu.sync_copy(zz_h, agg_sh.at[pl.ds(sid * NPS, NPS)])
        plsc.subcore_barrier()

        def chunk(kk, carry):
            o = kk * CH
            pltpu.sync_copy(m2_h.at[pl.ds(base + o, CH)], m_v)
            pltpu.sync_copy(dst_h.at[pl.ds(base + o, CH)], idx_v)
            pltpu.sync_copy(m_v, agg_sh.at[idx_v], add=True)
            return carry

        lax.fori_loop(0, NCH, chunk, 0)
        plsc.subcore_barrier()
        pltpu.sync_copy(agg_sh.at[pl.ds(sid * NPS, NPS)],
                        out_h.at[cid, pl.ds(sid * NPS, NPS)])

    return k(m2, dst, zeros_n)


BRN = 2000
BRE = 2000


def _tc_prep0(x, ws, wd):
    def body(x_ref, ws_ref, wd_ref, a_ref, b_ref):
        xv = x_ref[...]
        a_ref[...] = jnp.dot(xv, ws_ref[...], preferred_element_type=F32)
        b_ref[...] = jnp.dot(xv, wd_ref[...], preferred_element_type=F32)

    return pl.pallas_call(
        body,
        grid=(N // BRN,),
        in_specs=[
            pl.BlockSpec((BRN, H), lambda i: (i, 0)),
            pl.BlockSpec((H, H), lambda i: (0, 0)),
            pl.BlockSpec((H, H), lambda i: (0, 0)),
        ],
        out_specs=[pl.BlockSpec((BRN, H), lambda i: (i, 0))] * 2,
        out_shape=[jax.ShapeDtypeStruct((N, H), F32)] * 2,
    )(x, ws, wd)


def _tc_edge(g, ea, wea, we2, prm):
    def body(g_ref, ea_ref, wea_ref, we2_ref, prm_ref, o_ref):
        m1 = (g_ref[...]
              + jnp.dot(ea_ref[...], wea_ref[...], preferred_element_type=F32)
              + prm_ref[0:1, :])
        m1 = m1 * lax.logistic(m1)
        m2 = jnp.dot(m1, we2_ref[...], preferred_element_type=F32) + prm_ref[1:2, :]
        o_ref[...] = m2 * lax.logistic(m2)

    return pl.pallas_call(
        body,
        grid=(E // BRE,),
        in_specs=[
            pl.BlockSpec((BRE, H), lambda i: (i, 0)),
            pl.BlockSpec((BRE, ED), lambda i: (i, 0)),
            pl.BlockSpec((ED, H), lambda i: (0, 0)),
            pl.BlockSpec((H, H), lambda i: (0, 0)),
            pl.BlockSpec((8, H), lambda i: (0, 0)),
        ],
        out_specs=pl.BlockSpec((BRE, H), lambda i: (i, 0)),
        out_shape=jax.ShapeDtypeStruct((E, H), F32),
    )(g, ea, wea, we2, prm)


def _tc_node(h, p0, p1, wn1h, wn1a, wn2, prm, ws=None, wd=None):
    with_prep = ws is not None

    def body(h_ref, p0_ref, p1_ref, wn1h_ref, wn1a_ref, wn2_ref, prm_ref,
             *rest):
        agg = p0_ref[...] + p1_ref[...]
        t = (jnp.dot(h_ref[...], wn1h_ref[...], preferred_element_type=F32)
             + jnp.dot(agg, wn1a_ref[...], preferred_element_type=F32)
             + prm_ref[0:1, :])
        t = t * lax.logistic(t)
        hn = jnp.dot(t, wn2_ref[...], preferred_element_type=F32) + prm_ref[1:2, :]
        if with_prep:
            ws_ref, wd_ref, hn_ref, a_ref, b_ref = rest
            hn_ref[...] = hn
            a_ref[...] = jnp.dot(hn, ws_ref[...], preferred_element_type=F32)
            b_ref[...] = jnp.dot(hn, wd_ref[...], preferred_element_type=F32)
        else:
            (hn_ref,) = rest
            hn_ref[...] = hn

    full = lambda i: (0, 0)
    blk = lambda i: (i, 0)
    in_specs = [
        pl.BlockSpec((BRN, H), blk),
        pl.BlockSpec((BRN, H), blk),
        pl.BlockSpec((BRN, H), blk),
        pl.BlockSpec((H, H), full),
        pl.BlockSpec((H, H), full),
        pl.BlockSpec((H, H), full),
        pl.BlockSpec((8, H), full),
    ]
    args = [h, p0, p1, wn1h, wn1a, wn2, prm]
    if with_prep:
        in_specs += [pl.BlockSpec((H, H), full)] * 2
        args += [ws, wd]
        out_specs = [pl.BlockSpec((BRN, H), blk)] * 3
        out_shape = [jax.ShapeDtypeStruct((N, H), F32)] * 3
    else:
        out_specs = pl.BlockSpec((BRN, H), blk)
        out_shape = jax.ShapeDtypeStruct((N, H), F32)

    return pl.pallas_call(
        body,
        grid=(N // BRN,),
        in_specs=in_specs,
        out_specs=out_specs,
        out_shape=out_shape,
    )(*args)


def kernel(x, pos, edge_index, edge_attr, We1, be1, We2, be2, Wn1, bn1, Wn2, bn2):
    src = edge_index[0]
    dst = edge_index[1]
    px = pos[:, 0]
    py = pos[:, 1]
    pz = pos[:, 2]
    zeros_t = jnp.zeros((NPS, H), F32)

    a, b = _tc_prep0(x, We1[0, :H], We1[0, H:2 * H])
    h = x
    for l in range(L):
        g = _gather_pass(a, b, src, dst, px, py, pz, We1[l, 2 * H])
        eprm = jnp.zeros((8, H), F32).at[0].set(be1[l]).at[1].set(be2[l])
        m2 = _tc_edge(g, edge_attr, We1[l, 2 * H + 1:], We2[l], eprm)
        aggp = _scatter_pass(m2, dst, zeros_t)
        p0 = aggp[0, :N]
        p1 = aggp[1, :N]
        nprm = jnp.zeros((8, H), F32).at[0].set(bn1[l]).at[1].set(bn2[l])
        if l < L - 1:
            h, a, b = _tc_node(h, p0, p1, Wn1[l, :H], Wn1[l, H:],
                               Wn2[l], nprm,
                               We1[l + 1, :H], We1[l + 1, H:2 * H])
        else:
            h = _tc_node(h, p0, p1, Wn1[l, :H], Wn1[l, H:],
                         Wn2[l], nprm)
    return h, pos


# speedup vs baseline: 3.7566x; 1.6173x over previous
import functools

import jax
import jax.numpy as jnp
from jax import lax
from jax.experimental import pallas as pl
from jax.experimental.pallas import tpu as pltpu
from jax.experimental.pallas import tpu_sc as plsc

N = 10000
E = 320000
H = 128
ED = 16
L = 2

NC = 2
NS = 16
NW = NC * NS
EPW = E // NW
CH = 80
NCH = EPW // CH
NPAD = 10240
NPS = NPAD // NS

F32 = jnp.float32
I32 = jnp.int32


def _sc_mesh():
    return plsc.VectorSubcoreMesh(
        core_axis_name="c", subcore_axis_name="s",
        num_cores=NC, num_subcores=NS)


def _gather_pass(a, b, src, dst, px, py, pz, wd2):
    @functools.partial(
        pl.kernel,
        out_type=jax.ShapeDtypeStruct((E, H), F32),
        mesh=_sc_mesh(),
        scratch_types=[
            pltpu.VMEM((EPW,), I32),
            pltpu.VMEM((EPW,), I32),
            pltpu.VMEM((N,), F32),
            pltpu.VMEM((N,), F32),
            pltpu.VMEM((N,), F32),
            pltpu.VMEM((H,), F32),
            pltpu.VMEM((CH, H), F32),
            pltpu.VMEM((CH, H), F32),
            pltpu.VMEM((CH, H), F32),
            pltpu.VMEM((CH, H), F32),
            pltpu.SemaphoreType.DMA,
            pltpu.SemaphoreType.DMA,
            pltpu.SemaphoreType.DMA,
            pltpu.SemaphoreType.DMA,
            pltpu.SemaphoreType.DMA,
            pltpu.SemaphoreType.DMA,
        ],
        compiler_params=pltpu.CompilerParams(needs_layout_passes=False),
    )
    def k(a_h, b_h, src_h, dst_h, px_h, py_h, pz_h, w_h, out_h,
          src_v, dst_v, px_v, py_v, pz_v, w_v,
          av0, av1, bv0, bv1, sga0, sga1, sgb0, sgb1, so0, so1):
        av = (av0, av1)
        bv = (bv0, bv1)
        sga = (sga0, sga1)
        sgb = (sgb0, sgb1)
        so = (so0, so1)
        cid = lax.axis_index("c")
        sid = lax.axis_index("s")
        base = (cid * NS + sid) * EPW
        pltpu.sync_copy(src_h.at[pl.ds(base, EPW)], src_v)
        pltpu.sync_copy(dst_h.at[pl.ds(base, EPW)], dst_v)
        pltpu.sync_copy(px_h, px_v)
        pltpu.sync_copy(py_h, py_v)
        pltpu.sync_copy(pz_h, pz_v)
        pltpu.sync_copy(w_h, w_v)
        wrow = [w_v[pl.ds(j * 16, 16)] for j in range(H // 16)]

        def gat_a(c, bf):
            return pltpu.make_async_copy(
                a_h.at[src_v.at[pl.ds(c * CH, CH)]], av[bf], sga[bf])

        def gat_b(c, bf):
            return pltpu.make_async_copy(
                b_h.at[dst_v.at[pl.ds(c * CH, CH)]], bv[bf], sgb[bf])

        def owr(c, bf):
            return pltpu.make_async_copy(
                av[bf], out_h.at[pl.ds(base + c * CH, CH)], so[bf])

        def issue(c, bf):
            gat_a(c, bf).start()
            gat_b(c, bf).start()

        def compute(c, bf):
            o = c * CH
            for g in range(CH // 16):
                r0 = g * 16
                si = src_v[pl.ds(o + r0, 16)]
                di = dst_v[pl.ds(o + r0, 16)]
                rx = plsc.load_gather(px_v, [si]) - plsc.load_gather(px_v, [di])
                ry = plsc.load_gather(py_v, [si]) - plsc.load_gather(py_v, [di])
                rz = plsc.load_gather(pz_v, [si]) - plsc.load_gather(pz_v, [di])
                d2g = rx * rx + ry * ry + rz * rz
                for i in range(16):
                    r = r0 + i
                    d2s = d2g[i]
                    for j in range(H // 16):
                        sl = pl.ds(j * 16, 16)
                        av[bf][r, sl] = av[bf][r, sl] + bv[bf][r, sl] + d2s * wrow[j]

        issue(0, 0)
        gat_a(0, 0).wait()
        gat_b(0, 0).wait()
        issue(1, 1)
        compute(0, 0)
        owr(0, 0).start()

        def pair(q, carry):
            c0 = 1 + 2 * q
            for bf, c in ((1, c0), (0, c0 + 1)):
                gat_a(c, bf).wait()
                gat_b(c, bf).wait()
                ob = 1 - bf

                @pl.when(c + 1 < NCH)
                def _():
                    owr(c - 1, ob).wait()
                    issue(c + 1, ob)

                compute(c, bf)
                owr(c, bf).start()
            return carry

        lax.fori_loop(0, (NCH - 1) // 2, pair, 0)
        owr(NCH - 2, 1).wait()
        owr(NCH - 1, 0).wait()

    return k(a, b, src, dst, px, py, pz, wd2)


def _scatter_pass(m2, dst, zeros_n):
    @functools.partial(
        pl.kernel,
        out_type=jax.ShapeDtypeStruct((NC, NPAD, H), F32),
        mesh=_sc_mesh(),
        scratch_types=[
            pltpu.VMEM((CH, H), F32),
            pltpu.VMEM((CH, H), F32),
            pltpu.VMEM((CH,), I32),
            pltpu.VMEM((CH,), I32),
            pltpu.VMEM_SHARED((NPAD, H), F32),
            pltpu.SemaphoreType.DMA,
            pltpu.SemaphoreType.DMA,
            pltpu.SemaphoreType.DMA,
            pltpu.SemaphoreType.DMA,
            pltpu.SemaphoreType.DMA,
            pltpu.SemaphoreType.DMA,
        ],
    )
    def k(m2_h, dst_h, zz_h, out_h, mv0, mv1, iv0, iv1, agg_sh,
          sm0, sm1, si0, si1, ss0, ss1):
        mv = (mv0, mv1)
        iv = (iv0, iv1)
        sm = (sm0, sm1)
        si = (si0, si1)
        ss = (ss0, ss1)
        cid = lax.axis_index("c")
        sid = lax.axis_index("s")
        base = (cid * NS + sid) * EPW
        pltpu.sync_copy(zz_h, agg_sh.at[pl.ds(sid * NPS, NPS)])
        plsc.subcore_barrier()

        def ld_m(c, bf):
            return pltpu.make_async_copy(
                m2_h.at[pl.ds(base + c * CH, CH)], mv[bf], sm[bf])

        def ld_i(c, bf):
            return pltpu.make_async_copy(
                dst_h.at[pl.ds(base + c * CH, CH)], iv[bf], si[bf])

        def scat(bf):
            return pltpu.make_async_copy(mv[bf], agg_sh.at[iv[bf]], ss[bf])

        def issue(c, bf):
            ld_m(c, bf).start()
            ld_i(c, bf).start()

        issue(0, 0)
        ld_m(0, 0).wait()
        ld_i(0, 0).wait()
        issue(1, 1)
        scat(0).start(add=True)

        def pair(q, carry):
            c0 = 1 + 2 * q
            for bf, c in ((1, c0), (0, c0 + 1)):
                ld_m(c, bf).wait()
                ld_i(c, bf).wait()
                ob = 1 - bf

                @pl.when(c + 1 < NCH)
                def _():
                    scat(ob).wait()
                    issue(c + 1, ob)

                scat(bf).start(add=True)
            return carry

        lax.fori_loop(0, (NCH - 1) // 2, pair, 0)
        scat(1).wait()
        scat(0).wait()
        plsc.subcore_barrier()
        pltpu.sync_copy(agg_sh.at[pl.ds(sid * NPS, NPS)],
                        out_h.at[cid, pl.ds(sid * NPS, NPS)])

    return k(m2, dst, zeros_n)


BRN = 2000
BRE = 2000


def _tc_prep0(x, ws, wd):
    def body(x_ref, ws_ref, wd_ref, a_ref, b_ref):
        xv = x_ref[...]
        a_ref[...] = jnp.dot(xv, ws_ref[...], preferred_element_type=F32)
        b_ref[...] = jnp.dot(xv, wd_ref[...], preferred_element_type=F32)

    return pl.pallas_call(
        body,
        grid=(N // BRN,),
        in_specs=[
            pl.BlockSpec((BRN, H), lambda i: (i, 0)),
            pl.BlockSpec((H, H), lambda i: (0, 0)),
            pl.BlockSpec((H, H), lambda i: (0, 0)),
        ],
        out_specs=[pl.BlockSpec((BRN, H), lambda i: (i, 0))] * 2,
        out_shape=[jax.ShapeDtypeStruct((N, H), F32)] * 2,
    )(x, ws, wd)


def _tc_edge(g, ea, wea, we2, prm):
    def body(g_ref, ea_ref, wea_ref, we2_ref, prm_ref, o_ref):
        m1 = (g_ref[...]
              + jnp.dot(ea_ref[...], wea_ref[...], preferred_element_type=F32)
              + prm_ref[0:1, :])
        m1 = m1 * lax.logistic(m1)
        m2 = jnp.dot(m1, we2_ref[...], preferred_element_type=F32) + prm_ref[1:2, :]
        o_ref[...] = m2 * lax.logistic(m2)

    return pl.pallas_call(
        body,
        grid=(E // BRE,),
        in_specs=[
            pl.BlockSpec((BRE, H), lambda i: (i, 0)),
            pl.BlockSpec((BRE, ED), lambda i: (i, 0)),
            pl.BlockSpec((ED, H), lambda i: (0, 0)),
            pl.BlockSpec((H, H), lambda i: (0, 0)),
            pl.BlockSpec((8, H), lambda i: (0, 0)),
        ],
        out_specs=pl.BlockSpec((BRE, H), lambda i: (i, 0)),
        out_shape=jax.ShapeDtypeStruct((E, H), F32),
    )(g, ea, wea, we2, prm)


def _tc_node(h, p0, p1, wn1h, wn1a, wn2, prm, ws=None, wd=None):
    with_prep = ws is not None

    def body(h_ref, p0_ref, p1_ref, wn1h_ref, wn1a_ref, wn2_ref, prm_ref,
             *rest):
        agg = p0_ref[...] + p1_ref[...]
        t = (jnp.dot(h_ref[...], wn1h_ref[...], preferred_element_type=F32)
             + jnp.dot(agg, wn1a_ref[...], preferred_element_type=F32)
             + prm_ref[0:1, :])
        t = t * lax.logistic(t)
        hn = jnp.dot(t, wn2_ref[...], preferred_element_type=F32) + prm_ref[1:2, :]
        if with_prep:
            ws_ref, wd_ref, hn_ref, a_ref, b_ref = rest
            hn_ref[...] = hn
            a_ref[...] = jnp.dot(hn, ws_ref[...], preferred_element_type=F32)
            b_ref[...] = jnp.dot(hn, wd_ref[...], preferred_element_type=F32)
        else:
            (hn_ref,) = rest
            hn_ref[...] = hn

    full = lambda i: (0, 0)
    blk = lambda i: (i, 0)
    in_specs = [
        pl.BlockSpec((BRN, H), blk),
        pl.BlockSpec((BRN, H), blk),
        pl.BlockSpec((BRN, H), blk),
        pl.BlockSpec((H, H), full),
        pl.BlockSpec((H, H), full),
        pl.BlockSpec((H, H), full),
        pl.BlockSpec((8, H), full),
    ]
    args = [h, p0, p1, wn1h, wn1a, wn2, prm]
    if with_prep:
        in_specs += [pl.BlockSpec((H, H), full)] * 2
        args += [ws, wd]
        out_specs = [pl.BlockSpec((BRN, H), blk)] * 3
        out_shape = [jax.ShapeDtypeStruct((N, H), F32)] * 3
    else:
        out_specs = pl.BlockSpec((BRN, H), blk)
        out_shape = jax.ShapeDtypeStruct((N, H), F32)

    return pl.pallas_call(
        body,
        grid=(N // BRN,),
        in_specs=in_specs,
        out_specs=out_specs,
        out_shape=out_shape,
    )(*args)


def kernel(x, pos, edge_index, edge_attr, We1, be1, We2, be2, Wn1, bn1, Wn2, bn2):
    src = edge_index[0]
    dst = edge_index[1]
    px = pos[:, 0]
    py = pos[:, 1]
    pz = pos[:, 2]
    zeros_t = jnp.zeros((NPS, H), F32)

    a, b = _tc_prep0(x, We1[0, :H], We1[0, H:2 * H])
    h = x
    for l in range(L):
        g = _gather_pass(a, b, src, dst, px, py, pz, We1[l, 2 * H])
        eprm = jnp.zeros((8, H), F32).at[0].set(be1[l]).at[1].set(be2[l])
        m2 = _tc_edge(g, edge_attr, We1[l, 2 * H + 1:], We2[l], eprm)
        aggp = _scatter_pass(m2, dst, zeros_t)
        p0 = aggp[0, :N]
        p1 = aggp[1, :N]
        nprm = jnp.zeros((8, H), F32).at[0].set(bn1[l]).at[1].set(bn2[l])
        if l < L - 1:
            h, a, b = _tc_node(h, p0, p1, Wn1[l, :H], Wn1[l, H:],
                               Wn2[l], nprm,
                               We1[l + 1, :H], We1[l + 1, H:2 * H])
        else:
            h = _tc_node(h, p0, p1, Wn1[l, :H], Wn1[l, H:],
                         Wn2[l], nprm)
    return h, pos


# speedup vs baseline: 5.1839x; 1.3799x over previous
import functools

import jax
import jax.numpy as jnp
from jax import lax
from jax.experimental import pallas as pl
from jax.experimental.pallas import tpu as pltpu
from jax.experimental.pallas import tpu_sc as plsc

N = 10000
E = 320000
H = 128
ED = 16
L = 2

NC = 2
NS = 16
NW = NC * NS
EPW = E // NW
CH = 80
NCH = EPW // CH
NPAD = 10240
NPS = NPAD // NS

F32 = jnp.float32
I32 = jnp.int32


def _sc_mesh():
    return plsc.VectorSubcoreMesh(
        core_axis_name="c", subcore_axis_name="s",
        num_cores=NC, num_subcores=NS)


NB = 3
GHEAD = 5


def _gather_pass(a, b, src, dst, px, py, pz, wd2):
    @functools.partial(
        pl.kernel,
        out_type=jax.ShapeDtypeStruct((E, H), F32),
        mesh=_sc_mesh(),
        scratch_types=[
            pltpu.VMEM((N,), F32),
            pltpu.VMEM((N,), F32),
            pltpu.VMEM((N,), F32),
            pltpu.VMEM((H,), F32),
            [pltpu.VMEM((CH, H), F32)] * 2,
            [pltpu.VMEM((CH, H), F32)] * 2,
            [pltpu.VMEM((CH, H), F32)] * 2,
            [pltpu.VMEM((CH,), I32)] * 4,
            [pltpu.VMEM((CH,), I32)] * 4,
            [pltpu.SemaphoreType.DMA] * 2,
            [pltpu.SemaphoreType.DMA] * 2,
            [pltpu.SemaphoreType.DMA] * 2,
            [pltpu.SemaphoreType.DMA] * 4,
            [pltpu.SemaphoreType.DMA] * 4,
        ],
        compiler_params=pltpu.CompilerParams(needs_layout_passes=False),
    )
    def k(a_h, b_h, src_h, dst_h, px_h, py_h, pz_h, w_h, out_h,
          px_v, py_v, pz_v, w_v, av, bv, gv, ivs, ivd,
          sga, sgb, so, sis, sid_):
        cid = lax.axis_index("c")
        sid = lax.axis_index("s")
        base = (cid * NS + sid) * EPW
        pltpu.sync_copy(px_h, px_v)
        pltpu.sync_copy(py_h, py_v)
        pltpu.sync_copy(pz_h, pz_v)
        pltpu.sync_copy(w_h, w_v)
        wrow = [w_v[pl.ds(j * 16, 16)] for j in range(H // 16)]

        def ld_is(c, bi):
            return pltpu.make_async_copy(
                src_h.at[pl.ds(base + c * CH, CH)], ivs[bi], sis[bi])

        def ld_id(c, bi):
            return pltpu.make_async_copy(
                dst_h.at[pl.ds(base + c * CH, CH)], ivd[bi], sid_[bi])

        def gat_a(bf, bi):
            return pltpu.make_async_copy(a_h.at[ivs[bi]], av[bf], sga[bf])

        def gat_b(bf, bi):
            return pltpu.make_async_copy(b_h.at[ivd[bi]], bv[bf], sgb[bf])

        def owr(c, bf):
            return pltpu.make_async_copy(
                gv[bf], out_h.at[pl.ds(base + c * CH, CH)], so[bf])

        def compute(bf, bi):
            def grp(g, carry):
                r0 = g * 16
                si = ivs[bi][pl.ds(r0, 16)]
                di = ivd[bi][pl.ds(r0, 16)]
                rx = plsc.load_gather(px_v, [si]) - plsc.load_gather(px_v, [di])
                ry = plsc.load_gather(py_v, [si]) - plsc.load_gather(py_v, [di])
                rz = plsc.load_gather(pz_v, [si]) - plsc.load_gather(pz_v, [di])
                d2g = rx * rx + ry * ry + rz * rz
                for i in range(16):
                    r = r0 + i
                    d2s = d2g[i]
                    for j in range(H // 16):
                        sl = pl.ds(j * 16, 16)
                        gv[bf][r, sl] = (av[bf][r, sl] + bv[bf][r, sl]
                                         + d2s * wrow[j])
                return carry

            lax.fori_loop(0, CH // 16, grp, 0)

        def step(c, bf, bi):
            bf1, bi1, bi2 = 1 - bf, (bi + 1) % 4, (bi + 2) % 4
            gat_a(bf, bi).wait()
            gat_b(bf, bi).wait()

            def ahead1():
                ld_is(c + 1, bi1).wait()
                ld_id(c + 1, bi1).wait()
                gat_a(bf1, bi1).start()
                gat_b(bf1, bi1).start()

            def ahead2():
                ld_is(c + 2, bi2).start()
                ld_id(c + 2, bi2).start()

            if isinstance(c, int):
                if c + 1 < NCH:
                    ahead1()
                if c + 2 < NCH:
                    ahead2()
                if c >= 2:
                    owr(c - 2, bf).wait()
            else:
                pl.when(c + 1 < NCH)(ahead1)
                pl.when(c + 2 < NCH)(ahead2)
                owr(c - 2, bf).wait()

            compute(bf, bi)
            owr(c, bf).start()

        ld_is(0, 0).start()
        ld_id(0, 0).start()
        ld_is(1, 1).start()
        ld_id(1, 1).start()
        ld_is(0, 0).wait()
        ld_id(0, 0).wait()
        gat_a(0, 0).start()
        gat_b(0, 0).start()
        for c in range(GHEAD):
            step(c, c % 2, c % 4)

        def body(q, carry):
            c0 = GHEAD + 4 * q
            for r in range(4):
                cr = GHEAD + r
                step(c0 + r, cr % 2, cr % 4)
            return carry

        lax.fori_loop(0, (NCH - GHEAD) // 4, body, 0)
        owr(NCH - 2, (NCH - 2) % 2).wait()
        owr(NCH - 1, (NCH - 1) % 2).wait()

    return k(a, b, src, dst, px, py, pz, wd2)


def _scatter_pass(m2, dst, zeros_n):
    @functools.partial(
        pl.kernel,
        out_type=jax.ShapeDtypeStruct((NC, NPAD, H), F32),
        mesh=_sc_mesh(),
        scratch_types=[
            [pltpu.VMEM((CH, H), F32)] * NB,
            [pltpu.VMEM((CH,), I32)] * NB,
            pltpu.VMEM_SHARED((NPAD, H), F32),
            [pltpu.SemaphoreType.DMA] * NB,
            [pltpu.SemaphoreType.DMA] * NB,
            [pltpu.SemaphoreType.DMA] * NB,
        ],
    )
    def k(m2_h, dst_h, zz_h, out_h, mv, iv, agg_sh, sm, si, ss):
        cid = lax.axis_index("c")
        sid = lax.axis_index("s")
        base = (cid * NS + sid) * EPW
        pltpu.sync_copy(zz_h, agg_sh.at[pl.ds(sid * NPS, NPS)])
        plsc.subcore_barrier()

        def ld_m(c, bf):
            return pltpu.make_async_copy(
                m2_h.at[pl.ds(base + c * CH, CH)], mv[bf], sm[bf])

        def ld_i(c, bf):
            return pltpu.make_async_copy(
                dst_h.at[pl.ds(base + c * CH, CH)], iv[bf], si[bf])

        def scat(bf):
            return pltpu.make_async_copy(mv[bf], agg_sh.at[iv[bf]], ss[bf])

        def issue(c, bf):
            ld_m(c, bf).start()
            ld_i(c, bf).start()

        def step(c, bf):
            ld_m(c, bf).wait()
            ld_i(c, bf).wait()
            nb = (bf + NB - 1) % NB

            if isinstance(c, int):
                if c + NB - 1 < NCH:
                    if c >= 1:
                        scat(nb).wait()
                    issue(c + NB - 1, nb)
            else:
                @pl.when(c + NB - 1 < NCH)
                def _():
                    scat(nb).wait()
                    issue(c + NB - 1, nb)

            scat(bf).start(add=True)

        for c in range(NB - 1):
            issue(c, c)
        for c in range(NB - 1):
            step(c, c)

        def body(q, carry):
            c0 = (NB - 1) + NB * q
            for r in range(NB):
                step(c0 + r, (NB - 1 + r) % NB)
            return carry

        lax.fori_loop(0, (NCH - (NB - 1)) // NB, body, 0)
        for bf in range(NB):
            scat(bf).wait()
        plsc.subcore_barrier()
        pltpu.sync_copy(agg_sh.at[pl.ds(sid * NPS, NPS)],
                        out_h.at[cid, pl.ds(sid * NPS, NPS)])

    return k(m2, dst, zeros_n)


BRN = 2000
BRE = 2000


def _tc_prep0(x, ws, wd):
    def body(x_ref, ws_ref, wd_ref, a_ref, b_ref):
        xv = x_ref[...]
        a_ref[...] = jnp.dot(xv, ws_ref[...], preferred_element_type=F32)
        b_ref[...] = jnp.dot(xv, wd_ref[...], preferred_element_type=F32)

    return pl.pallas_call(
        body,
        grid=(N // BRN,),
        in_specs=[
            pl.BlockSpec((BRN, H), lambda i: (i, 0)),
            pl.BlockSpec((H, H), lambda i: (0, 0)),
            pl.BlockSpec((H, H), lambda i: (0, 0)),
        ],
        out_specs=[pl.BlockSpec((BRN, H), lambda i: (i, 0))] * 2,
        out_shape=[jax.ShapeDtypeStruct((N, H), F32)] * 2,
    )(x, ws, wd)


def _tc_edge(g, ea, wea, we2, prm):
    def body(g_ref, ea_ref, wea_ref, we2_ref, prm_ref, o_ref):
        m1 = (g_ref[...]
              + jnp.dot(ea_ref[...], wea_ref[...], preferred_element_type=F32)
              + prm_ref[0:1, :])
        m1 = m1 * lax.logistic(m1)
        m2 = jnp.dot(m1, we2_ref[...], preferred_element_type=F32) + prm_ref[1:2, :]
        o_ref[...] = m2 * lax.logistic(m2)

    return pl.pallas_call(
        body,
        grid=(E // BRE,),
        in_specs=[
            pl.BlockSpec((BRE, H), lambda i: (i, 0)),
            pl.BlockSpec((BRE, ED), lambda i: (i, 0)),
            pl.BlockSpec((ED, H), lambda i: (0, 0)),
            pl.BlockSpec((H, H), lambda i: (0, 0)),
            pl.BlockSpec((8, H), lambda i: (0, 0)),
        ],
        out_specs=pl.BlockSpec((BRE, H), lambda i: (i, 0)),
        out_shape=jax.ShapeDtypeStruct((E, H), F32),
    )(g, ea, wea, we2, prm)


def _tc_node(h, aggp, wn1h, wn1a, wn2, prm, ws=None, wd=None):
    with_prep = ws is not None

    def body(h_ref, ag_ref, wn1h_ref, wn1a_ref, wn2_ref, prm_ref,
             *rest):
        agg = ag_ref[0] + ag_ref[1]
        t = (jnp.dot(h_ref[...], wn1h_ref[...], preferred_element_type=F32)
             + jnp.dot(agg, wn1a_ref[...], preferred_element_type=F32)
             + prm_ref[0:1, :])
        t = t * lax.logistic(t)
        hn = jnp.dot(t, wn2_ref[...], preferred_element_type=F32) + prm_ref[1:2, :]
        if with_prep:
            ws_ref, wd_ref, hn_ref, a_ref, b_ref = rest
            hn_ref[...] = hn
            a_ref[...] = jnp.dot(hn, ws_ref[...], preferred_element_type=F32)
            b_ref[...] = jnp.dot(hn, wd_ref[...], preferred_element_type=F32)
        else:
            (hn_ref,) = rest
            hn_ref[...] = hn

    full = lambda i: (0, 0)
    blk = lambda i: (i, 0)
    in_specs = [
        pl.BlockSpec((BRN, H), blk),
        pl.BlockSpec((NC, BRN, H), lambda i: (0, i, 0)),
        pl.BlockSpec((H, H), full),
        pl.BlockSpec((H, H), full),
        pl.BlockSpec((H, H), full),
        pl.BlockSpec((8, H), full),
    ]
    args = [h, aggp, wn1h, wn1a, wn2, prm]
    if with_prep:
        in_specs += [pl.BlockSpec((H, H), full)] * 2
        args += [ws, wd]
        out_specs = [pl.BlockSpec((BRN, H), blk)] * 3
        out_shape = [jax.ShapeDtypeStruct((N, H), F32)] * 3
    else:
        out_specs = pl.BlockSpec((BRN, H), blk)
        out_shape = jax.ShapeDtypeStruct((N, H), F32)

    return pl.pallas_call(
        body,
        grid=(N // BRN,),
        in_specs=in_specs,
        out_specs=out_specs,
        out_shape=out_shape,
    )(*args)


def kernel(x, pos, edge_index, edge_attr, We1, be1, We2, be2, Wn1, bn1, Wn2, bn2):
    src = edge_index[0]
    dst = edge_index[1]
    px = pos[:, 0]
    py = pos[:, 1]
    pz = pos[:, 2]
    zeros_t = jnp.zeros((NPS, H), F32)

    a, b = _tc_prep0(x, We1[0, :H], We1[0, H:2 * H])
    h = x
    for l in range(L):
        g = _gather_pass(a, b, src, dst, px, py, pz, We1[l, 2 * H])
        eprm = jnp.zeros((8, H), F32).at[0].set(be1[l]).at[1].set(be2[l])
        m2 = _tc_edge(g, edge_attr, We1[l, 2 * H + 1:], We2[l], eprm)
        aggp = _scatter_pass(m2, dst, zeros_t)
        nprm = jnp.zeros((8, H), F32).at[0].set(bn1[l]).at[1].set(bn2[l])
        if l < L - 1:
            h, a, b = _tc_node(h, aggp, Wn1[l, :H], Wn1[l, H:],
                               Wn2[l], nprm,
                               We1[l + 1, :H], We1[l + 1, H:2 * H])
        else:
            h = _tc_node(h, aggp, Wn1[l, :H], Wn1[l, H:],
                         Wn2[l], nprm)
    return h, pos
